# Initial kernel scaffold; baseline (speedup 1.0000x reference)
import jax
import jax.numpy as jnp
from jax.experimental import pallas as pl


def kernel(x, edge_attr, e1w1, e1b1, e1w2, e1b2, e1w3, e1b3, root1, cbias1, bn1g, bn1b, e2w1, e2b1, e2w2, e2b2, e2w3, e2b3, root2, cbias2, bn2g, bn2b, fc1w, fc1b, bn3g, bn3b, fc2w, fc2b, fc3w, fc3b, edge_index, batch):
    raise NotImplementedError("write your pallas kernel here")


# speedup vs baseline: 1.3330x; 1.3330x over previous
import functools

import jax
import jax.numpy as jnp
from jax import lax
from jax.experimental import pallas as pl
from jax.experimental.pallas import tpu as pltpu
from jax.experimental.pallas import tpu_sc as plsc

_N = 5000
_E = 20000
_NF = 4
_NG = 32
_NPAD = 5120
_EPAD = 20480
_NW = 32
_CHUNK = 128
_NCHUNK = 5
_EPW = _NCHUNK * _CHUNK
_STRIPE = _NPAD // 16

_B1 = 2048
_B2 = 512


def _sc_mesh():
    return plsc.VectorSubcoreMesh(core_axis_name="c", subcore_axis_name="s")


def _sc_gather(table, idx3, d):
    @functools.partial(
        pl.kernel,
        out_type=jax.ShapeDtypeStruct((_EPAD, d), jnp.float32),
        mesh=_sc_mesh(),
        scratch_types=[
            pltpu.VMEM((_NCHUNK, _CHUNK), jnp.int32),
            pltpu.VMEM((_CHUNK, d), jnp.float32),
            pltpu.SemaphoreType.DMA,
        ],
    )
    def k(table_hbm, idx_hbm, out_hbm, idx_v, rows_v, sem):
        c = lax.axis_index("c")
        s = lax.axis_index("s")
        wid = s * 2 + c
        pltpu.sync_copy(idx_hbm.at[wid], idx_v)
        for j in range(_NCHUNK):
            pltpu.async_copy(table_hbm.at[idx_v.at[j]], rows_v, sem).wait()
            pltpu.sync_copy(
                rows_v, out_hbm.at[pl.ds(wid * _EPW + j * _CHUNK, _CHUNK)])

    return k(table, idx3)


def _sc_scatter_add(rows, idx3, zeros_init, d):
    @functools.partial(
        pl.kernel,
        out_type=jax.ShapeDtypeStruct((2, _NPAD, d), jnp.float32),
        mesh=_sc_mesh(),
        scratch_types=[
            pltpu.VMEM((_NCHUNK, _CHUNK), jnp.int32),
            pltpu.VMEM((_EPW, d), jnp.float32),
            pltpu.VMEM_SHARED((_NPAD, d), jnp.float32),
        ],
    )
    def k(rows_hbm, idx_hbm, zeros_hbm, out_hbm, idx_v, rows_v, acc_sh):
        c = lax.axis_index("c")
        s = lax.axis_index("s")
        wid = s * 2 + c
        pltpu.sync_copy(zeros_hbm.at[pl.ds(s * _STRIPE, _STRIPE)],
                        acc_sh.at[pl.ds(s * _STRIPE, _STRIPE)])
        plsc.subcore_barrier()
        pltpu.sync_copy(idx_hbm.at[wid], idx_v)
        pltpu.sync_copy(rows_hbm.at[pl.ds(wid * _EPW, _EPW)], rows_v)
        for j in range(_NCHUNK):
            pltpu.sync_copy(rows_v.at[pl.ds(j * _CHUNK, _CHUNK)],
                            acc_sh.at[idx_v.at[j]], add=True)
        plsc.subcore_barrier()
        pltpu.sync_copy(acc_sh.at[pl.ds(s * _STRIPE, _STRIPE)],
                        out_hbm.at[c, pl.ds(s * _STRIPE, _STRIPE)])

    return k(rows, idx3, zeros_init)


def _tc_messages1(ea, xg, w1, b1, w2, b2, w3, b3):
    def body(ea_ref, xg_ref, w1r, b1r, w2r, b2r, w3r, b3r, out_ref):
        h = jnp.maximum(ea_ref[...] @ w1r[...] + b1r[...], 0.0)
        h = jnp.maximum(h @ w2r[...] + b2r[...], 0.0)
        z = jnp.maximum(h @ w3r[...] + b3r[...], 0.0)
        xgv = xg_ref[...]
        acc = xgv[:, 0:1] * z[:, 0:128]
        for i in range(1, _NF):
            acc = acc + xgv[:, i:i + 1] * z[:, i * 128:(i + 1) * 128]
        out_ref[...] = acc

    full = lambda shape: pl.BlockSpec(shape, lambda i: (0, 0))
    return pl.pallas_call(
        body,
        grid=(_EPAD // _B1,),
        in_specs=[
            pl.BlockSpec((_B1, 6), lambda i: (i, 0)),
            pl.BlockSpec((_B1, 128), lambda i: (i, 0)),
            full((6, 64)), full((1, 64)),
            full((64, 64)), full((1, 64)),
            full((64, _NF * 128)), full((1, _NF * 128)),
        ],
        out_specs=pl.BlockSpec((_B1, 128), lambda i: (i, 0)),
        out_shape=jax.ShapeDtypeStruct((_EPAD, 128), jnp.float32),
    )(ea, xg, w1, b1, w2, b2, w3, b3)


def _tc_messages2(ea, hg, w1, b1, w2, b2, w3, b3):
    def body(ea_ref, hg_ref, w1r, b1r, w2r, b2r, w3r, b3r, out_ref):
        h = jnp.maximum(ea_ref[...] @ w1r[...] + b1r[...], 0.0)
        h = jnp.maximum(h @ w2r[...] + b2r[...], 0.0)
        z = jnp.maximum(h @ w3r[...] + b3r[...], 0.0)
        hgv = hg_ref[...]
        acc = hgv[:, 0:1] * z[:, 0:64]
        for i in range(1, 128):
            acc = acc + hgv[:, i:i + 1] * z[:, i * 64:(i + 1) * 64]
        out_ref[...] = jnp.concatenate(
            [acc, jnp.zeros((acc.shape[0], 64), jnp.float32)], axis=1)

    full = lambda shape: pl.BlockSpec(shape, lambda i: (0, 0))
    return pl.pallas_call(
        body,
        grid=(_EPAD // _B2,),
        in_specs=[
            pl.BlockSpec((_B2, 6), lambda i: (i, 0)),
            pl.BlockSpec((_B2, 128), lambda i: (i, 0)),
            full((6, 64)), full((1, 64)),
            full((64, 64)), full((1, 64)),
            full((64, 128 * 64)), full((1, 128 * 64)),
        ],
        out_specs=pl.BlockSpec((_B2, 128), lambda i: (i, 0)),
        out_shape=jax.ShapeDtypeStruct((_EPAD, 128), jnp.float32),
    )(ea, hg, w1, b1, w2, b2, w3, b3)


def _tc_node_update1(parts, x, root1, cbias1, bn1g, bn1b):
    def body(p_ref, x_ref, rr, cb, gr, br, out_ref):
        agg = p_ref[0, :_N, :] + p_ref[1, :_N, :]
        h = jnp.maximum(agg + x_ref[...] @ rr[...] + cb[...], 0.0)
        scale = gr[...] * jax.lax.rsqrt(jnp.float32(1.0 + 1e-5))
        out_ref[...] = h * scale + br[...]

    full = lambda shape: pl.BlockSpec(shape, lambda: tuple(0 for _ in shape))
    return pl.pallas_call(
        body,
        in_specs=[
            full((2, _NPAD, 128)),
            full((_N, _NF)),
            full((_NF, 128)), full((1, 128)),
            full((1, 128)), full((1, 128)),
        ],
        out_specs=full((_N, 128)),
        out_shape=jax.ShapeDtypeStruct((_N, 128), jnp.float32),
    )(parts, x, root1, cbias1, bn1g, bn1b)


def _tc_head(parts, h, root2, cbias2, bn2g, bn2b, batch2d,
             fc1w, fc1b, bn3g, bn3b, fc2w, fc2b, fc3w, fc3b):
    def body(p_ref, h_ref, rr, cb, g2, b2, batch_ref,
             w1, bb1, g3, b3, w2, bb2, w3, bb3, out_ref):
        hv = h_ref[...]
        agg = p_ref[0, :_N, :64] + p_ref[1, :_N, :64]
        h2 = jnp.maximum(agg + hv @ rr[...] + cb[...], 0.0)
        inv = jax.lax.rsqrt(jnp.float32(1.0 + 1e-5))
        h2 = h2 * (g2[...] * inv) + b2[...]
        rows = lax.broadcasted_iota(jnp.int32, (_NG, _N), 0)
        onehot = jnp.where(rows == batch_ref[...], 1.0, 0.0)
        g = onehot @ h2
        g = jnp.maximum(g @ w1[...] + bb1[...], 0.0)
        g = g * (g3[...] * inv) + b3[...]
        g = jnp.maximum(g @ w2[...] + bb2[...], 0.0)
        logits = g @ w3[...] + bb3[...]
        m = jnp.max(logits, axis=1, keepdims=True)
        lse = jnp.log(jnp.sum(jnp.exp(logits - m), axis=1, keepdims=True)) + m
        out_ref[...] = logits - lse

    full = lambda shape: pl.BlockSpec(shape, lambda: tuple(0 for _ in shape))
    return pl.pallas_call(
        body,
        in_specs=[
            full((2, _NPAD, 128)),
            full((_N, 128)),
            full((128, 64)), full((1, 64)),
            full((1, 64)), full((1, 64)),
            full((1, _N)),
            full((64, 64)), full((1, 64)),
            full((1, 64)), full((1, 64)),
            full((64, 64)), full((1, 64)),
            full((64, 2)), full((1, 2)),
        ],
        out_specs=full((_NG, 2)),
        out_shape=jax.ShapeDtypeStruct((_NG, 2), jnp.float32),
    )(parts, h, root2, cbias2, bn2g, bn2b, batch2d,
      fc1w, fc1b, bn3g, bn3b, fc2w, fc2b, fc3w, fc3b)


def kernel(x, edge_attr, e1w1, e1b1, e1w2, e1b2, e1w3, e1b3, root1, cbias1,
           bn1g, bn1b, e2w1, e2b1, e2w2, e2b2, e2w3, e2b3, root2, cbias2,
           bn2g, bn2b, fc1w, fc1b, bn3g, bn3b, fc2w, fc2b, fc3w, fc3b,
           edge_index, batch):
    src = edge_index[0]
    dst = edge_index[1]
    pad_e = _EPAD - _E
    src3 = jnp.pad(src, (0, pad_e)).reshape(_NW, _NCHUNK, _CHUNK)
    dst3 = jnp.pad(dst, (0, pad_e), constant_values=_N).reshape(
        _NW, _NCHUNK, _CHUNK)
    ea = jnp.pad(edge_attr, ((0, pad_e), (0, 0)))
    x128 = jnp.pad(x, ((0, 0), (0, 128 - _NF)))
    z128 = jnp.zeros((_NPAD, 128), jnp.float32)
    row = lambda v: v.reshape(1, -1)

    xg = _sc_gather(x128, src3, 128)
    m1 = _tc_messages1(ea, xg, e1w1, row(e1b1), e1w2, row(e1b2),
                       e1w3, row(e1b3))
    parts1 = _sc_scatter_add(m1, dst3, z128, 128)
    h = _tc_node_update1(parts1, x, root1, row(cbias1),
                         row(bn1g), row(bn1b))

    hg = _sc_gather(h, src3, 128)
    m2 = _tc_messages2(ea, hg, e2w1, row(e2b1), e2w2, row(e2b2),
                       e2w3, row(e2b3))
    parts2 = _sc_scatter_add(m2, dst3, z128, 128)
    return _tc_head(parts2, h, root2, row(cbias2), row(bn2g), row(bn2b),
                    batch.reshape(1, _N).astype(jnp.int32),
                    fc1w, row(fc1b), row(bn3g), row(bn3b),
                    fc2w, row(fc2b), fc3w, row(fc3b))


# speedup vs baseline: 3.3542x; 2.5163x over previous
import functools

import jax
import jax.numpy as jnp
from jax import lax
from jax.experimental import pallas as pl
from jax.experimental.pallas import tpu as pltpu
from jax.experimental.pallas import tpu_sc as plsc

_N = 5000
_E = 20000
_NF = 4
_NG = 32
_NPAD = 5120
_EPAD = 20480
_NW = 32
_CHUNK = 128
_NCHUNK = 5
_EPW = _NCHUNK * _CHUNK
_STRIPE = _NPAD // 16

_B1 = 2048
_B2 = 512


def _sc_mesh():
    return plsc.VectorSubcoreMesh(core_axis_name="c", subcore_axis_name="s")


def _sc_gather(table, idx3, d):
    @functools.partial(
        pl.kernel,
        out_type=jax.ShapeDtypeStruct((_EPAD, d), jnp.float32),
        mesh=_sc_mesh(),
        scratch_types=[
            pltpu.VMEM((_NCHUNK, _CHUNK), jnp.int32),
            pltpu.VMEM((_CHUNK, d), jnp.float32),
            pltpu.SemaphoreType.DMA,
        ],
    )
    def k(table_hbm, idx_hbm, out_hbm, idx_v, rows_v, sem):
        c = lax.axis_index("c")
        s = lax.axis_index("s")
        wid = s * 2 + c
        pltpu.sync_copy(idx_hbm.at[wid], idx_v)
        for j in range(_NCHUNK):
            pltpu.async_copy(table_hbm.at[idx_v.at[j]], rows_v, sem).wait()
            pltpu.sync_copy(
                rows_v, out_hbm.at[pl.ds(wid * _EPW + j * _CHUNK, _CHUNK)])

    return k(table, idx3)


def _sc_scatter_add(rows, idx3, zeros_init, d):
    @functools.partial(
        pl.kernel,
        out_type=jax.ShapeDtypeStruct((2, _NPAD, d), jnp.float32),
        mesh=_sc_mesh(),
        scratch_types=[
            pltpu.VMEM((_NCHUNK, _CHUNK), jnp.int32),
            pltpu.VMEM((_EPW, d), jnp.float32),
            pltpu.VMEM_SHARED((_NPAD, d), jnp.float32),
        ],
    )
    def k(rows_hbm, idx_hbm, zeros_hbm, out_hbm, idx_v, rows_v, acc_sh):
        c = lax.axis_index("c")
        s = lax.axis_index("s")
        wid = s * 2 + c
        pltpu.sync_copy(zeros_hbm.at[pl.ds(s * _STRIPE, _STRIPE)],
                        acc_sh.at[pl.ds(s * _STRIPE, _STRIPE)])
        plsc.subcore_barrier()
        pltpu.sync_copy(idx_hbm.at[wid], idx_v)
        pltpu.sync_copy(rows_hbm.at[pl.ds(wid * _EPW, _EPW)], rows_v)
        for j in range(_NCHUNK):
            pltpu.sync_copy(rows_v.at[pl.ds(j * _CHUNK, _CHUNK)],
                            acc_sh.at[idx_v.at[j]], add=True)
        plsc.subcore_barrier()
        pltpu.sync_copy(acc_sh.at[pl.ds(s * _STRIPE, _STRIPE)],
                        out_hbm.at[c, pl.ds(s * _STRIPE, _STRIPE)])

    return k(rows, idx3, zeros_init)


def _tc_messages1(ea, xg, w1, b1, w2, b2, w3, b3):
    def body(ea_ref, xg_ref, w1r, b1r, w2r, b2r, w3r, b3r, out_ref):
        h = jnp.maximum(ea_ref[...] @ w1r[...] + b1r[...], 0.0)
        h = jnp.maximum(h @ w2r[...] + b2r[...], 0.0)
        z = jnp.maximum(h @ w3r[...] + b3r[...], 0.0)
        xgv = xg_ref[...]
        acc = xgv[:, 0:1] * z[:, 0:128]
        for i in range(1, _NF):
            acc = acc + xgv[:, i:i + 1] * z[:, i * 128:(i + 1) * 128]
        out_ref[...] = acc

    full = lambda shape: pl.BlockSpec(shape, lambda i: (0, 0))
    return pl.pallas_call(
        body,
        grid=(_EPAD // _B1,),
        in_specs=[
            pl.BlockSpec((_B1, 6), lambda i: (i, 0)),
            pl.BlockSpec((_B1, 128), lambda i: (i, 0)),
            full((6, 64)), full((1, 64)),
            full((64, 64)), full((1, 64)),
            full((64, _NF * 128)), full((1, _NF * 128)),
        ],
        out_specs=pl.BlockSpec((_B1, 128), lambda i: (i, 0)),
        out_shape=jax.ShapeDtypeStruct((_EPAD, 128), jnp.float32),
    )(ea, xg, w1, b1, w2, b2, w3, b3)


def _tc_messages2(ea, hg, w1, b1, w2, b2, w3aug):
    def body(ea_ref, hg_ref, w1r, b1r, w2r, b2r, w3r, out_ref):
        h = jnp.maximum(ea_ref[...] @ w1r[...] + b1r[...], 0.0)
        h = jnp.maximum(h @ w2r[...] + b2r[...], 0.0)
        h_aug = jnp.concatenate(
            [h.T, jnp.ones((1, _B2), jnp.float32)], axis=0)
        z_t = jnp.maximum(w3r[...] @ h_aug, 0.0)
        hg_t = hg_ref[...].T
        acc = hg_t[0:1, :] * z_t[0:64, :]
        for i in range(1, 128):
            acc = acc + hg_t[i:i + 1, :] * z_t[i * 64:(i + 1) * 64, :]
        out_ref[...] = jnp.concatenate(
            [acc.T, jnp.zeros((_B2, 64), jnp.float32)], axis=1)

    full = lambda shape: pl.BlockSpec(shape, lambda i: (0, 0))
    return pl.pallas_call(
        body,
        grid=(_EPAD // _B2,),
        in_specs=[
            pl.BlockSpec((_B2, 6), lambda i: (i, 0)),
            pl.BlockSpec((_B2, 128), lambda i: (i, 0)),
            full((6, 64)), full((1, 64)),
            full((64, 64)), full((1, 64)),
            full((128 * 64, 65)),
        ],
        out_specs=pl.BlockSpec((_B2, 128), lambda i: (i, 0)),
        out_shape=jax.ShapeDtypeStruct((_EPAD, 128), jnp.float32),
    )(ea, hg, w1, b1, w2, b2, w3aug)


def _tc_node_update1(parts, x, root1, cbias1, bn1g, bn1b):
    def body(p_ref, x_ref, rr, cb, gr, br, out_ref):
        agg = p_ref[0, :_N, :] + p_ref[1, :_N, :]
        h = jnp.maximum(agg + x_ref[...] @ rr[...] + cb[...], 0.0)
        scale = gr[...] * jax.lax.rsqrt(jnp.float32(1.0 + 1e-5))
        out_ref[...] = h * scale + br[...]

    full = lambda shape: pl.BlockSpec(shape, lambda: tuple(0 for _ in shape))
    return pl.pallas_call(
        body,
        in_specs=[
            full((2, _NPAD, 128)),
            full((_N, _NF)),
            full((_NF, 128)), full((1, 128)),
            full((1, 128)), full((1, 128)),
        ],
        out_specs=full((_N, 128)),
        out_shape=jax.ShapeDtypeStruct((_N, 128), jnp.float32),
    )(parts, x, root1, cbias1, bn1g, bn1b)


def _tc_head(parts, h, root2, cbias2, bn2g, bn2b, batch2d,
             fc1w, fc1b, bn3g, bn3b, fc2w, fc2b, fc3w, fc3b):
    def body(p_ref, h_ref, rr, cb, g2, b2, batch_ref,
             w1, bb1, g3, b3, w2, bb2, w3, bb3, out_ref):
        hv = h_ref[...]
        agg = p_ref[0, :_N, :64] + p_ref[1, :_N, :64]
        h2 = jnp.maximum(agg + hv @ rr[...] + cb[...], 0.0)
        inv = jax.lax.rsqrt(jnp.float32(1.0 + 1e-5))
        h2 = h2 * (g2[...] * inv) + b2[...]
        rows = lax.broadcasted_iota(jnp.int32, (_NG, _N), 0)
        onehot = jnp.where(rows == batch_ref[...], 1.0, 0.0)
        g = onehot @ h2
        g = jnp.maximum(g @ w1[...] + bb1[...], 0.0)
        g = g * (g3[...] * inv) + b3[...]
        g = jnp.maximum(g @ w2[...] + bb2[...], 0.0)
        logits = g @ w3[...] + bb3[...]
        m = jnp.max(logits, axis=1, keepdims=True)
        lse = jnp.log(jnp.sum(jnp.exp(logits - m), axis=1, keepdims=True)) + m
        out_ref[...] = logits - lse

    full = lambda shape: pl.BlockSpec(shape, lambda: tuple(0 for _ in shape))
    return pl.pallas_call(
        body,
        in_specs=[
            full((2, _NPAD, 128)),
            full((_N, 128)),
            full((128, 64)), full((1, 64)),
            full((1, 64)), full((1, 64)),
            full((1, _N)),
            full((64, 64)), full((1, 64)),
            full((1, 64)), full((1, 64)),
            full((64, 64)), full((1, 64)),
            full((64, 2)), full((1, 2)),
        ],
        out_specs=full((_NG, 2)),
        out_shape=jax.ShapeDtypeStruct((_NG, 2), jnp.float32),
    )(parts, h, root2, cbias2, bn2g, bn2b, batch2d,
      fc1w, fc1b, bn3g, bn3b, fc2w, fc2b, fc3w, fc3b)


def kernel(x, edge_attr, e1w1, e1b1, e1w2, e1b2, e1w3, e1b3, root1, cbias1,
           bn1g, bn1b, e2w1, e2b1, e2w2, e2b2, e2w3, e2b3, root2, cbias2,
           bn2g, bn2b, fc1w, fc1b, bn3g, bn3b, fc2w, fc2b, fc3w, fc3b,
           edge_index, batch):
    src = edge_index[0]
    dst = edge_index[1]
    pad_e = _EPAD - _E
    src3 = jnp.pad(src, (0, pad_e)).reshape(_NW, _NCHUNK, _CHUNK)
    dst3 = jnp.pad(dst, (0, pad_e), constant_values=_N).reshape(
        _NW, _NCHUNK, _CHUNK)
    ea = jnp.pad(edge_attr, ((0, pad_e), (0, 0)))
    x128 = jnp.pad(x, ((0, 0), (0, 128 - _NF)))
    z128 = jnp.zeros((_NPAD, 128), jnp.float32)
    row = lambda v: v.reshape(1, -1)

    xg = _sc_gather(x128, src3, 128)
    m1 = _tc_messages1(ea, xg, e1w1, row(e1b1), e1w2, row(e1b2),
                       e1w3, row(e1b3))
    parts1 = _sc_scatter_add(m1, dst3, z128, 128)
    h = _tc_node_update1(parts1, x, root1, row(cbias1),
                         row(bn1g), row(bn1b))

    hg = _sc_gather(h, src3, 128)
    w3aug = jnp.concatenate([e2w3.T, e2b3.reshape(-1, 1)], axis=1)
    m2 = _tc_messages2(ea, hg, e2w1, row(e2b1), e2w2, row(e2b2),
                       w3aug)
    parts2 = _sc_scatter_add(m2, dst3, z128, 128)
    return _tc_head(parts2, h, root2, row(cbias2), row(bn2g), row(bn2b),
                    batch.reshape(1, _N).astype(jnp.int32),
                    fc1w, row(fc1b), row(bn3g), row(bn3b),
                    fc2w, row(fc2b), fc3w, row(fc3b))


# speedup vs baseline: 3.9519x; 1.1782x over previous
import functools

import jax
import jax.numpy as jnp
from jax import lax
from jax.experimental import pallas as pl
from jax.experimental.pallas import tpu as pltpu
from jax.experimental.pallas import tpu_sc as plsc

_N = 5000
_E = 20000
_NF = 4
_NG = 32
_NPAD = 5120
_EPAD = 20480
_NW = 32
_CHUNK = 128
_NCHUNK = 5
_EPW = _NCHUNK * _CHUNK
_STRIPE = _NPAD // 16

_B1 = 2048
_B2 = 512


def _sc_mesh():
    return plsc.VectorSubcoreMesh(core_axis_name="c", subcore_axis_name="s")


def _sc_gather(table, idx3, d):
    @functools.partial(
        pl.kernel,
        out_type=jax.ShapeDtypeStruct((_EPAD, d), jnp.float32),
        mesh=_sc_mesh(),
        scratch_types=[
            pltpu.VMEM((_NCHUNK, _CHUNK), jnp.int32),
            pltpu.VMEM((_EPW, d), jnp.float32),
            pltpu.SemaphoreType.DMA,
        ],
    )
    def k(table_hbm, idx_hbm, out_hbm, idx_v, rows_v, sem):
        c = lax.axis_index("c")
        s = lax.axis_index("s")
        wid = s * 2 + c
        pltpu.sync_copy(idx_hbm.at[wid], idx_v)
        copies = [
            pltpu.async_copy(table_hbm.at[idx_v.at[j]],
                             rows_v.at[pl.ds(j * _CHUNK, _CHUNK)], sem)
            for j in range(_NCHUNK)
        ]
        for cp in copies:
            cp.wait()
        pltpu.sync_copy(rows_v, out_hbm.at[pl.ds(wid * _EPW, _EPW)])

    return k(table, idx3)


def _sc_gather_x(x_flat, idx3):
    @functools.partial(
        pl.kernel,
        out_type=jax.ShapeDtypeStruct((_NF, _EPAD), jnp.float32),
        mesh=_sc_mesh(),
        scratch_types=[
            pltpu.VMEM((_NF * _N,), jnp.float32),
            pltpu.VMEM((_NCHUNK, _CHUNK), jnp.int32),
            pltpu.VMEM((_NF, _EPW), jnp.float32),
        ],
        compiler_params=pltpu.CompilerParams(needs_layout_passes=False),
    )
    def k(x_hbm, idx_hbm, out_hbm, x_v, idx_v, xg_v):
        c = lax.axis_index("c")
        s = lax.axis_index("s")
        wid = s * 2 + c
        pltpu.sync_copy(x_hbm, x_v)
        pltpu.sync_copy(idx_hbm.at[wid], idx_v)
        for j in range(_NCHUNK):
            for g16 in range(_CHUNK // 16):
                src16 = idx_v[j, pl.ds(g16 * 16, 16)]
                base4 = src16 * _NF
                for i in range(_NF):
                    vals = plsc.load_gather(x_v, [base4 + i])
                    xg_v[i, pl.ds(j * _CHUNK + g16 * 16, 16)] = vals
        pltpu.sync_copy(xg_v, out_hbm.at[:, pl.ds(wid * _EPW, _EPW)])

    return k(x_flat, idx3)


def _sc_scatter_add(rows, idx3, zeros_init, d):
    @functools.partial(
        pl.kernel,
        out_type=jax.ShapeDtypeStruct((2, _NPAD, d), jnp.float32),
        mesh=_sc_mesh(),
    scratch_types=[
            pltpu.VMEM((_NCHUNK, _CHUNK), jnp.int32),
            pltpu.VMEM((_EPW, d), jnp.float32),
            pltpu.VMEM_SHARED((_NPAD, d), jnp.float32),
            pltpu.SemaphoreType.DMA,
            pltpu.SemaphoreType.DMA,
        ],
    )
    def k(rows_hbm, idx_hbm, zeros_hbm, out_hbm, idx_v, rows_v, acc_sh,
          sem, sem2):
        c = lax.axis_index("c")
        s = lax.axis_index("s")
        wid = s * 2 + c
        z_cp = pltpu.async_copy(zeros_hbm.at[pl.ds(s * _STRIPE, _STRIPE)],
                                acc_sh.at[pl.ds(s * _STRIPE, _STRIPE)], sem2)
        r_cp = pltpu.async_copy(rows_hbm.at[pl.ds(wid * _EPW, _EPW)],
                                rows_v, sem)
        pltpu.sync_copy(idx_hbm.at[wid], idx_v)
        z_cp.wait()
        r_cp.wait()
        plsc.subcore_barrier()
        adds = [
            pltpu.async_copy(rows_v.at[pl.ds(j * _CHUNK, _CHUNK)],
                             acc_sh.at[idx_v.at[j]], sem, add=True)
            for j in range(_NCHUNK)
        ]
        for cp in adds:
            cp.wait()
        plsc.subcore_barrier()
        pltpu.sync_copy(acc_sh.at[pl.ds(s * _STRIPE, _STRIPE)],
                        out_hbm.at[c, pl.ds(s * _STRIPE, _STRIPE)])

    return k(rows, idx3, zeros_init)


def _tc_messages1(ea, xgt, w1, b1, w2, b2, w3aug):
    def body(ea_ref, xgt_ref, w1r, b1r, w2r, b2r, w3r, out_ref):
        h = jnp.maximum(ea_ref[...] @ w1r[...] + b1r[...], 0.0)
        h = jnp.maximum(h @ w2r[...] + b2r[...], 0.0)
        h_aug = jnp.concatenate(
            [h.T, jnp.ones((1, _B1), jnp.float32)], axis=0)
        z_t = jnp.maximum(w3r[...] @ h_aug, 0.0)
        xgt = xgt_ref[...]
        acc = xgt[0:1, :] * z_t[0:128, :]
        for i in range(1, _NF):
            acc = acc + xgt[i:i + 1, :] * z_t[i * 128:(i + 1) * 128, :]
        out_ref[...] = acc.T

    full = lambda shape: pl.BlockSpec(shape, lambda i: (0, 0))
    return pl.pallas_call(
        body,
        grid=(_EPAD // _B1,),
        in_specs=[
            pl.BlockSpec((_B1, 6), lambda i: (i, 0)),
            pl.BlockSpec((_NF, _B1), lambda i: (0, i)),
            full((6, 64)), full((1, 64)),
            full((64, 64)), full((1, 64)),
            full((_NF * 128, 65)),
        ],
        out_specs=pl.BlockSpec((_B1, 128), lambda i: (i, 0)),
        out_shape=jax.ShapeDtypeStruct((_EPAD, 128), jnp.float32),
    )(ea, xgt, w1, b1, w2, b2, w3aug)


def _tc_messages2(ea, hg, w1, b1, w2, b2, w3aug):
    def body(ea_ref, hg_ref, w1r, b1r, w2r, b2r, w3r, out_ref):
        h = jnp.maximum(ea_ref[...] @ w1r[...] + b1r[...], 0.0)
        h = jnp.maximum(h @ w2r[...] + b2r[...], 0.0)
        h_aug = jnp.concatenate(
            [h.T, jnp.ones((1, _B2), jnp.float32)], axis=0)
        z_t = jnp.maximum(w3r[...] @ h_aug, 0.0)
        hg_t = hg_ref[...].T
        acc = hg_t[0:1, :] * z_t[0:64, :]
        for i in range(1, 128):
            acc = acc + hg_t[i:i + 1, :] * z_t[i * 64:(i + 1) * 64, :]
        out_ref[...] = jnp.concatenate(
            [acc.T, jnp.zeros((_B2, 64), jnp.float32)], axis=1)

    full = lambda shape: pl.BlockSpec(shape, lambda i: (0, 0))
    return pl.pallas_call(
        body,
        grid=(_EPAD // _B2,),
        in_specs=[
            pl.BlockSpec((_B2, 6), lambda i: (i, 0)),
            pl.BlockSpec((_B2, 128), lambda i: (i, 0)),
            full((6, 64)), full((1, 64)),
            full((64, 64)), full((1, 64)),
            full((128 * 64, 65)),
        ],
        out_specs=pl.BlockSpec((_B2, 128), lambda i: (i, 0)),
        out_shape=jax.ShapeDtypeStruct((_EPAD, 128), jnp.float32),
    )(ea, hg, w1, b1, w2, b2, w3aug)


def _tc_node_update1(parts, x, root1, cbias1, bn1g, bn1b):
    def body(p_ref, x_ref, rr, cb, gr, br, out_ref):
        agg = p_ref[0, :_N, :] + p_ref[1, :_N, :]
        h = jnp.maximum(agg + x_ref[...] @ rr[...] + cb[...], 0.0)
        scale = gr[...] * jax.lax.rsqrt(jnp.float32(1.0 + 1e-5))
        out_ref[...] = h * scale + br[...]

    full = lambda shape: pl.BlockSpec(shape, lambda: tuple(0 for _ in shape))
    return pl.pallas_call(
        body,
        in_specs=[
            full((2, _NPAD, 128)),
            full((_N, _NF)),
            full((_NF, 128)), full((1, 128)),
            full((1, 128)), full((1, 128)),
        ],
        out_specs=full((_N, 128)),
        out_shape=jax.ShapeDtypeStruct((_N, 128), jnp.float32),
    )(parts, x, root1, cbias1, bn1g, bn1b)


def _tc_head(parts, h, root2, cbias2, bn2g, bn2b, batch2d,
             fc1w, fc1b, bn3g, bn3b, fc2w, fc2b, fc3w, fc3b):
    def body(p_ref, h_ref, rr, cb, g2, b2, batch_ref,
             w1, bb1, g3, b3, w2, bb2, w3, bb3, out_ref):
        hv = h_ref[...]
        agg = p_ref[0, :_N, :64] + p_ref[1, :_N, :64]
        h2 = jnp.maximum(agg + hv @ rr[...] + cb[...], 0.0)
        inv = jax.lax.rsqrt(jnp.float32(1.0 + 1e-5))
        h2 = h2 * (g2[...] * inv) + b2[...]
        rows = lax.broadcasted_iota(jnp.int32, (_NG, _N), 0)
        onehot = jnp.where(rows == batch_ref[...], 1.0, 0.0)
        g = onehot @ h2
        g = jnp.maximum(g @ w1[...] + bb1[...], 0.0)
        g = g * (g3[...] * inv) + b3[...]
        g = jnp.maximum(g @ w2[...] + bb2[...], 0.0)
        logits = g @ w3[...] + bb3[...]
        m = jnp.max(logits, axis=1, keepdims=True)
        lse = jnp.log(jnp.sum(jnp.exp(logits - m), axis=1, keepdims=True)) + m
        out_ref[...] = logits - lse

    full = lambda shape: pl.BlockSpec(shape, lambda: tuple(0 for _ in shape))
    return pl.pallas_call(
        body,
        in_specs=[
            full((2, _NPAD, 128)),
            full((_N, 128)),
            full((128, 64)), full((1, 64)),
            full((1, 64)), full((1, 64)),
            full((1, _N)),
            full((64, 64)), full((1, 64)),
            full((1, 64)), full((1, 64)),
            full((64, 64)), full((1, 64)),
            full((64, 2)), full((1, 2)),
        ],
        out_specs=full((_NG, 2)),
        out_shape=jax.ShapeDtypeStruct((_NG, 2), jnp.float32),
    )(parts, h, root2, cbias2, bn2g, bn2b, batch2d,
      fc1w, fc1b, bn3g, bn3b, fc2w, fc2b, fc3w, fc3b)


def kernel(x, edge_attr, e1w1, e1b1, e1w2, e1b2, e1w3, e1b3, root1, cbias1,
           bn1g, bn1b, e2w1, e2b1, e2w2, e2b2, e2w3, e2b3, root2, cbias2,
           bn2g, bn2b, fc1w, fc1b, bn3g, bn3b, fc2w, fc2b, fc3w, fc3b,
           edge_index, batch):
    src = edge_index[0]
    dst = edge_index[1]
    pad_e = _EPAD - _E
    src3 = jnp.pad(src, (0, pad_e)).reshape(_NW, _NCHUNK, _CHUNK)
    dst3 = jnp.pad(dst, (0, pad_e), constant_values=_N).reshape(
        _NW, _NCHUNK, _CHUNK)
    ea = jnp.pad(edge_attr, ((0, pad_e), (0, 0)))
    z128 = jnp.zeros((_NPAD, 128), jnp.float32)
    row = lambda v: v.reshape(1, -1)

    xgt = _sc_gather_x(x.reshape(-1), src3)
    w3aug1 = jnp.concatenate([e1w3.T, e1b3.reshape(-1, 1)], axis=1)
    m1 = _tc_messages1(ea, xgt, e1w1, row(e1b1), e1w2, row(e1b2),
                       w3aug1)
    parts1 = _sc_scatter_add(m1, dst3, z128, 128)
    h = _tc_node_update1(parts1, x, root1, row(cbias1),
                         row(bn1g), row(bn1b))

    hg = _sc_gather(h, src3, 128)
    w3aug = jnp.concatenate([e2w3.T, e2b3.reshape(-1, 1)], axis=1)
    m2 = _tc_messages2(ea, hg, e2w1, row(e2b1), e2w2, row(e2b2),
                       w3aug)
    parts2 = _sc_scatter_add(m2, dst3, z128, 128)
    return _tc_head(parts2, h, root2, row(cbias2), row(bn2g), row(bn2b),
                    batch.reshape(1, _N).astype(jnp.int32),
                    fc1w, row(fc1b), row(bn3g), row(bn3b),
                    fc2w, row(fc2b), fc3w, row(fc3b))
